# baseline (speedup 1.0000x reference)
import functools

import jax
import jax.numpy as jnp
from jax import lax
from jax.experimental import pallas as pl
from jax.experimental.pallas import tpu as pltpu
from jax.experimental.pallas import tpu_sc as plsc

_T = 8192
_D = 768
_F = 3072
_E = 8

_TM = 512
_NTP = _T // _TM + _E
_TP = _NTP * _TM

_NC = 2
_NS = 16
_NW = _NC * _NS
_CH = 64
_GPW = _TP // _CH // _NW
_SPW = _T // _CH // _NW


def _router_body(x_ref, wr_ref, idx_ref):
    l = jnp.dot(x_ref[...], wr_ref[...], preferred_element_type=jnp.float32)
    idx_ref[0, 0, :] = jnp.argmax(l, axis=-1).astype(jnp.int32)


_RB = 1024


def _router(x, Wr):
    return pl.pallas_call(
        _router_body,
        grid=(_T // _RB,),
        in_specs=[
            pl.BlockSpec((_RB, _D), lambda i: (i, 0)),
            pl.BlockSpec((_D, _E), lambda i: (0, 0)),
        ],
        out_specs=pl.BlockSpec((1, 1, _RB), lambda i: (i, 0, 0)),
        out_shape=jax.ShapeDtypeStruct((_T // _RB, 1, _RB), jnp.int32),
    )(x, Wr)


def _layout(idx):
    iota_t = jnp.arange(_T, dtype=jnp.int32)
    idx_s, perm = lax.sort_key_val(idx, iota_t)
    goff = jnp.searchsorted(
        idx_s, jnp.arange(_E + 1, dtype=jnp.int32), side="left"
    ).astype(jnp.int32)
    counts = goff[1:] - goff[:-1]
    pc = (counts + _TM - 1) // _TM
    sc = jnp.cumsum(pc).astype(jnp.int32)
    s0 = jnp.concatenate([jnp.zeros((1,), jnp.int32), sc[:-1]])
    nuse = sc[_E - 1]

    tvec = jnp.arange(_NTP, dtype=jnp.int32)
    gid = jnp.clip(jnp.searchsorted(sc, tvec, side="right").astype(jnp.int32),
                   0, _E - 1)
    vld = (tvec < nuse).astype(jnp.int32)

    p = jnp.arange(_TP, dtype=jnp.int32)
    ep = jnp.repeat(gid, _TM)
    rel = p - s0[ep] * _TM
    real = (rel >= 0) & (rel < counts[ep])
    cpos = jnp.clip(goff[ep] + rel, 0, _T - 1)
    gidx = jnp.where(real, perm[cpos], 0).astype(jnp.int32)

    j = jnp.arange(_T, dtype=jnp.int32)
    ej = jnp.clip(jnp.searchsorted(goff, j, side="right").astype(jnp.int32) - 1,
                  0, _E - 1)
    ppos = (s0[ej] * _TM + (j - goff[ej])).astype(jnp.int32)

    return perm, gidx, ppos, gid, vld, jnp.maximum(nuse, 1)


def _ffn_body(gid_ref, vld_ref, nu_ref, xs_ref, wr_ref, w1_ref, w2_ref,
              out_ref):
    t = pl.program_id(0)

    @pl.when(vld_ref[t] > 0)
    def _():
        xv = xs_ref[...]
        l = jnp.dot(xv, wr_ref[...], preferred_element_type=jnp.float32)
        mx = jnp.max(l, axis=-1)
        gate = 1.0 / jnp.sum(jnp.exp(l - mx[:, None]), axis=-1)
        xg = xv * gate[:, None]
        h = jnp.maximum(
            jnp.dot(xg, w1_ref[0, :, :], preferred_element_type=jnp.float32),
            0.0)
        out_ref[...] = xv + jnp.dot(
            h, w2_ref[0, :, :], preferred_element_type=jnp.float32)


def _ffn(xs, Wr, W1, W2, gid, vld, nuse):
    def _tclamp(t, gid, vld, nu):
        return (jnp.minimum(t, nu[0] - 1), 0)

    grid_spec = pltpu.PrefetchScalarGridSpec(
        num_scalar_prefetch=3,
        grid=(_NTP,),
        in_specs=[
            pl.BlockSpec((_TM, _D), _tclamp),
            pl.BlockSpec((_D, _E), lambda t, gid, vld, nu: (0, 0)),
            pl.BlockSpec((1, _D, _F),
                         lambda t, gid, vld, nu: (gid[jnp.minimum(t, nu[0] - 1)], 0, 0)),
            pl.BlockSpec((1, _F, _D),
                         lambda t, gid, vld, nu: (gid[jnp.minimum(t, nu[0] - 1)], 0, 0)),
        ],
        out_specs=pl.BlockSpec((_TM, _D), _tclamp),
    )
    return pl.pallas_call(
        _ffn_body,
        grid_spec=grid_spec,
        out_shape=jax.ShapeDtypeStruct((_TP, _D), jnp.float32),
        compiler_params=pltpu.CompilerParams(
            dimension_semantics=("arbitrary",)),
    )(gid, vld, nuse.reshape(1), xs, Wr, W1, W2)


def _gather_body(x_hbm, gidx_hbm, xs_hbm, idx_m, rows_v, sem):
    wid = lax.axis_index("s") * _NC + lax.axis_index("c")
    pltpu.sync_copy(gidx_hbm.at[wid], idx_m)
    for c in range(_GPW):
        pltpu.async_copy(x_hbm.at[idx_m.at[c]], rows_v, sem).wait()
        pltpu.sync_copy(
            rows_v, xs_hbm.at[pl.ds((wid * _GPW + c) * _CH, _CH)])


@functools.cache
def _gather():
    mesh = plsc.VectorSubcoreMesh(core_axis_name="c", subcore_axis_name="s")
    return pl.kernel(
        _gather_body,
        out_type=jax.ShapeDtypeStruct((_TP, _D), jnp.float32),
        mesh=mesh,
        scratch_types=[
            pltpu.VMEM((_GPW, _CH), jnp.int32),
            pltpu.VMEM((_CH, _D), jnp.float32),
            pltpu.SemaphoreType.DMA,
        ],
    )


def _combine_body(ys_hbm, ppos_hbm, perm_hbm, out_hbm, pidx_m, oidx_m,
                  rows_v, s1, s2):
    wid = lax.axis_index("s") * _NC + lax.axis_index("c")
    pltpu.sync_copy(ppos_hbm.at[wid], pidx_m)
    pltpu.sync_copy(perm_hbm.at[wid], oidx_m)
    for c in range(_SPW):
        pltpu.async_copy(ys_hbm.at[pidx_m.at[c]], rows_v, s1).wait()
        pltpu.async_copy(rows_v, out_hbm.at[oidx_m.at[c]], s2).wait()


@functools.cache
def _combine():
    mesh = plsc.VectorSubcoreMesh(core_axis_name="c", subcore_axis_name="s")
    return pl.kernel(
        _combine_body,
        out_type=jax.ShapeDtypeStruct((_T, _D), jnp.float32),
        mesh=mesh,
        scratch_types=[
            pltpu.VMEM((_SPW, _CH), jnp.int32),
            pltpu.VMEM((_SPW, _CH), jnp.int32),
            pltpu.VMEM((_CH, _D), jnp.float32),
            pltpu.SemaphoreType.DMA,
            pltpu.SemaphoreType.DMA,
        ],
    )


def kernel(x, Wr, W1, W2):
    idx = _router(x, Wr).reshape(_T)
    perm, gidx, ppos, gid, vld, nuse = _layout(idx)
    xs = _gather()(x, gidx.reshape(_NW, _GPW, _CH))
    ys = _ffn(xs, Wr, W1, W2, gid, vld, nuse)
    return _combine()(ys, ppos.reshape(_NW, _SPW, _CH),
                      perm.reshape(_NW, _SPW, _CH))


# speedup vs baseline: 1.9481x; 1.9481x over previous
import functools

import jax
import jax.numpy as jnp
from jax import lax
from jax.experimental import pallas as pl
from jax.experimental.pallas import tpu as pltpu
from jax.experimental.pallas import tpu_sc as plsc

_T = 8192
_D = 768
_F = 3072
_E = 8

_TM = 512
_NTILES = _T // _TM
_NWORK = _NTILES + _E - 1

_NC = 2
_NS = 16
_NW = _NC * _NS
_RPW = _T // _NW
_CH = 64
_NCH = _RPW // _CH


def _router_body(x_ref, wr_ref, idx_ref):
    l = jnp.dot(x_ref[...], wr_ref[...], preferred_element_type=jnp.float32)
    idx_ref[0, 0, :] = jnp.argmax(l, axis=-1).astype(jnp.int32)


_RB = 1024


def _router(x, Wr):
    return pl.pallas_call(
        _router_body,
        grid=(_T // _RB,),
        in_specs=[
            pl.BlockSpec((_RB, _D), lambda i: (i, 0)),
            pl.BlockSpec((_D, _E), lambda i: (0, 0)),
        ],
        out_specs=pl.BlockSpec((1, 1, _RB), lambda i: (i, 0, 0)),
        out_shape=jax.ShapeDtypeStruct((_T // _RB, 1, _RB), jnp.int32),
    )(x, Wr)


def _metadata(goff):
    start, end = goff[:-1], goff[1:]
    nonempty = end > start
    first_t = start // _TM
    last_t = jnp.where(nonempty, (end - 1) // _TM, first_t)
    items = jnp.where(nonempty, last_t - first_t + 1, 0)
    ib = jnp.concatenate(
        [jnp.zeros((1,), jnp.int32), jnp.cumsum(items).astype(jnp.int32)])
    j = jnp.arange(_NWORK, dtype=jnp.int32)
    total = ib[_E]
    gsel = jnp.clip(
        jnp.searchsorted(ib, j, side="right").astype(jnp.int32) - 1, 0, _E - 1)
    tile = first_t[gsel] + (j - ib[gsel])
    valid = (j < total).astype(jnp.int32)
    tile = jnp.where(valid == 1, tile, _NTILES - 1).astype(jnp.int32)
    prev = jnp.concatenate([jnp.full((1,), -1, jnp.int32), tile[:-1]])
    ini = ((valid == 1) & (tile != prev)).astype(jnp.int32)
    return gsel.astype(jnp.int32), tile, valid, ini, goff


def _ffn_body(gid_ref, tid_ref, vld_ref, ini_ref, goff_ref,
              xs_ref, wr_ref, w1_ref, w2_ref, out_ref):
    i = pl.program_id(0)
    g = gid_ref[i]
    rows = tid_ref[i] * _TM + lax.broadcasted_iota(jnp.int32, (_TM, 1), 0)
    m = (rows >= goff_ref[g]) & (rows < goff_ref[g + 1]) & (vld_ref[i] > 0)
    l = jnp.dot(xs_ref[...], wr_ref[...], preferred_element_type=jnp.float32)
    mx = jnp.max(l, axis=-1)
    gate = 1.0 / jnp.sum(jnp.exp(l - mx[:, None]), axis=-1)
    xg = jnp.where(m, xs_ref[...] * gate[:, None], 0.0)
    h = jnp.maximum(
        jnp.dot(xg, w1_ref[0, :, :], preferred_element_type=jnp.float32), 0.0)
    c = jnp.dot(h, w2_ref[0, :, :], preferred_element_type=jnp.float32)

    @pl.when(ini_ref[i] > 0)
    def _():
        out_ref[...] = xs_ref[...] + c

    @pl.when(ini_ref[i] == 0)
    def _():
        out_ref[...] = out_ref[...] + c


def _ffn(xs, Wr, W1, W2, gid, tid, vld, ini, goff):
    grid_spec = pltpu.PrefetchScalarGridSpec(
        num_scalar_prefetch=5,
        grid=(_NWORK,),
        in_specs=[
            pl.BlockSpec((_TM, _D),
                         lambda i, gid, tid, vld, ini, goff: (tid[i], 0)),
            pl.BlockSpec((_D, _E),
                         lambda i, gid, tid, vld, ini, goff: (0, 0)),
            pl.BlockSpec((1, _D, _F),
                         lambda i, gid, tid, vld, ini, goff: (gid[i], 0, 0)),
            pl.BlockSpec((1, _F, _D),
                         lambda i, gid, tid, vld, ini, goff: (gid[i], 0, 0)),
        ],
        out_specs=pl.BlockSpec(
            (_TM, _D), lambda i, gid, tid, vld, ini, goff: (tid[i], 0)),
    )
    return pl.pallas_call(
        _ffn_body,
        grid_spec=grid_spec,
        out_shape=jax.ShapeDtypeStruct((_T, _D), jnp.float32),
        compiler_params=pltpu.CompilerParams(
            dimension_semantics=("arbitrary",)),
    )(gid, tid, vld, ini, goff, xs, Wr, W1, W2)


def _gather_body(x_hbm, perm_hbm, xs_hbm, idx_m, b0, b1, g0, g1, w0, w1):
    wid = lax.axis_index("s") * _NC + lax.axis_index("c")
    pltpu.sync_copy(perm_hbm.at[pl.ds(wid * _NCH, _NCH)], idx_m)
    bufs, gsems, wsems = (b0, b1), (g0, g1), (w0, w1)
    pend_g = [None, None]
    pend_w = [None, None]
    for c in range(_NCH):
        b = c % 2
        if pend_w[b] is not None:
            pend_w[b].wait()
        pend_g[b] = pltpu.async_copy(x_hbm.at[idx_m.at[c]], bufs[b], gsems[b])
        pend_g[b].wait()
        pend_w[b] = pltpu.async_copy(
            bufs[b], xs_hbm.at[pl.ds(wid * _RPW + c * _CH, _CH)], wsems[b])
    pend_w[(_NCH - 2) % 2].wait()
    pend_w[(_NCH - 1) % 2].wait()


@functools.cache
def _gather():
    mesh = plsc.VectorSubcoreMesh(core_axis_name="c", subcore_axis_name="s")
    return pl.kernel(
        _gather_body,
        out_type=jax.ShapeDtypeStruct((_T, _D), jnp.float32),
        mesh=mesh,
        scratch_types=[
            pltpu.VMEM((_NCH, _CH), jnp.int32),
            pltpu.VMEM((_CH, _D), jnp.float32),
            pltpu.VMEM((_CH, _D), jnp.float32),
            pltpu.SemaphoreType.DMA,
            pltpu.SemaphoreType.DMA,
            pltpu.SemaphoreType.DMA,
            pltpu.SemaphoreType.DMA,
        ],
    )


def _scatter_body(ys_hbm, perm_hbm, out_hbm, idx_m, b0, b1, g0, g1, w0, w1):
    wid = lax.axis_index("s") * _NC + lax.axis_index("c")
    pltpu.sync_copy(perm_hbm.at[pl.ds(wid * _NCH, _NCH)], idx_m)
    bufs, rsems, ssems = (b0, b1), (g0, g1), (w0, w1)
    pend_r = [None, None]
    pend_s = [None, None]
    for c in range(_NCH):
        b = c % 2
        if pend_s[b] is not None:
            pend_s[b].wait()
        pend_r[b] = pltpu.async_copy(
            ys_hbm.at[pl.ds(wid * _RPW + c * _CH, _CH)], bufs[b], rsems[b])
        pend_r[b].wait()
        pend_s[b] = pltpu.async_copy(bufs[b], out_hbm.at[idx_m.at[c]], ssems[b])
    pend_s[(_NCH - 2) % 2].wait()
    pend_s[(_NCH - 1) % 2].wait()


@functools.cache
def _scatter():
    mesh = plsc.VectorSubcoreMesh(core_axis_name="c", subcore_axis_name="s")
    return pl.kernel(
        _scatter_body,
        out_type=jax.ShapeDtypeStruct((_T, _D), jnp.float32),
        mesh=mesh,
        scratch_types=[
            pltpu.VMEM((_NCH, _CH), jnp.int32),
            pltpu.VMEM((_CH, _D), jnp.float32),
            pltpu.VMEM((_CH, _D), jnp.float32),
            pltpu.SemaphoreType.DMA,
            pltpu.SemaphoreType.DMA,
            pltpu.SemaphoreType.DMA,
            pltpu.SemaphoreType.DMA,
        ],
    )


def kernel(x, Wr, W1, W2):
    idx3 = _router(x, Wr)
    idx = idx3.reshape(_T)
    idx_s, perm = lax.sort_key_val(idx, jnp.arange(_T, dtype=jnp.int32))
    goff = jnp.searchsorted(
        idx_s, jnp.arange(_E + 1, dtype=jnp.int32), side="left"
    ).astype(jnp.int32)
    gid, tid, vld, ini, goff = _metadata(goff)
    perm2 = perm.reshape(_NW * _NCH, _CH)
    xs = _gather()(x, perm2)
    ys = _ffn(xs, Wr, W1, W2, gid, tid, vld, ini, goff)
    return _scatter()(ys, perm2)
